# baseline (speedup 1.0000x reference)
import jax
import jax.numpy as jnp
from jax import lax
from jax.experimental import pallas as pl
from jax.experimental.pallas import tpu as pltpu
from jax.experimental.pallas import tpu_sc as plsc

R = 128
N = 32768
K = 64
NC = 2
NS = 16
NW = NC * NS
ROWS_PER_W = R // NW
L = 16
NGRP = 32
STRIDE = NGRP * L
SEGLEN = N // STRIDE
NJ = SEGLEN // L
P1_UNROLL = 4

NEG_INF = float("-inf")


def _tec_body(x_hbm, out_hbm, buf0, buf1, buf2, outbuf, smax0, smax1, sem):
    wid = lax.axis_index("s") * NC + lax.axis_index("c")
    iota = lax.iota(jnp.int32, L)
    neg_vec = jnp.full((L,), NEG_INF, jnp.float32)
    lane0 = iota == 0
    row0 = wid * ROWS_PER_W
    bufs = [buf0, buf1, buf2]
    smaxs = [smax0, smax1]

    def phase1(rowbuf, smax):
        def p1_body(j, ms):
            ms = list(ms)
            for u in range(P1_UNROLL):
                base = pl.multiple_of((j * P1_UNROLL + u) * STRIDE, STRIDE)
                for g in range(NGRP):
                    ms[g] = jnp.maximum(ms[g], rowbuf[pl.ds(base + g * L, L)])
            return tuple(ms)

        init = tuple(jnp.full((L,), NEG_INF, jnp.float32) for _ in range(NGRP))
        segmax = lax.fori_loop(0, SEGLEN // P1_UNROLL, p1_body, init)
        t0 = jnp.full((L,), NEG_INF, jnp.float32)
        t1 = jnp.full((L,), NEG_INF, jnp.float32)
        for g in range(NGRP):
            smax[pl.ds(g * L, L)] = segmax[g]
            if g < L:
                t0 = jnp.where(iota == g, jnp.max(segmax[g]), t0)
            else:
                t1 = jnp.where(iota == g - L, jnp.max(segmax[g]), t1)
        return t0, t1

    def ext_round(i, t0, t1, smax, rowbuf, r):
        gmv = jnp.zeros((L,), jnp.float32) + jnp.max(jnp.maximum(t0, t1))
        f0 = plsc.all_reduce_ffs(t0 >= gmv) + jnp.zeros((L,), jnp.int32)
        f1 = plsc.all_reduce_ffs(t1 >= gmv) + jnp.zeros((L,), jnp.int32)
        in0 = f0 < L
        g_spl = jnp.where(in0, f0, f1 + L)
        gvec = plsc.load_gather(smax, [g_spl * L + iota])
        l_spl = plsc.all_reduce_ffs(gvec >= gmv) + jnp.zeros((L,), jnp.int32)
        base = g_spl * L + l_spl

        pmin = jnp.full((L,), 4096, jnp.int32)
        nm2 = neg_vec
        cl = jnp.zeros((L,), jnp.int32)
        for ja in range(NJ):
            idx = (ja * L + iota) * STRIDE + base
            v = plsc.load_gather(rowbuf, [idx])
            eq = v >= gmv
            jpos = ja * L + iota
            pmin = jnp.minimum(pmin, jnp.where(eq, jpos, 4096))
            nm2 = jnp.maximum(nm2, jnp.where(eq, neg_vec, v))
            cl = cl + jnp.where(eq, 1, 0)
        pos_vec = jnp.zeros((L,), jnp.int32) + jnp.min(pmin)
        kidx = pos_vec * STRIDE + base
        plsc.store_scatter(rowbuf, [kidx], neg_vec, mask=lane0)

        p1c = plsc.all_reduce_population_count(cl >= 1)
        p2c = plsc.all_reduce_population_count(cl >= 2)
        dup = (p1c >= 2) | (p2c >= 1)
        newmax = jnp.where(dup, gmv, jnp.zeros((L,), jnp.float32) + jnp.max(nm2))

        gvec2 = jnp.where(iota == l_spl, newmax, gvec)
        plsc.store_scatter(smax, [g_spl * L + iota], gvec2)
        nmx = jnp.zeros((L,), jnp.float32) + jnp.max(gvec2)
        t0 = jnp.where(in0 & (iota == g_spl), nmx, t0)
        t1 = jnp.where((~in0) & (iota == g_spl - L), nmx, t1)

        oidx = jnp.zeros((L,), jnp.int32) + i
        plsc.store_scatter(outbuf, [jnp.full((L,), r, jnp.int32), oidx],
                           gmv, mask=lane0)
        return t0, t1

    pltpu.sync_copy(x_hbm.at[row0], buf0)
    cp1 = pltpu.make_async_copy(x_hbm.at[row0 + 1], buf1, sem)
    cp1.start()
    cp3 = None

    for half in range(2):
        ra, rb = 2 * half, 2 * half + 1
        ta = phase1(bufs[ra % 3], smaxs[0])
        if half == 0:
            cp1.wait()
            cp2 = pltpu.make_async_copy(x_hbm.at[row0 + 2], buf2, sem)
            cp2.start()
        else:
            cp3.wait()
        tb = phase1(bufs[rb % 3], smaxs[1])

        def ext2_body(i, carry):
            ta0, ta1, tb0, tb1 = carry
            ta0, ta1 = ext_round(i, ta0, ta1, smaxs[0], bufs[ra % 3], ra)
            tb0, tb1 = ext_round(i, tb0, tb1, smaxs[1], bufs[rb % 3], rb)
            return ta0, ta1, tb0, tb1

        lax.fori_loop(0, K, ext2_body, (ta[0], ta[1], tb[0], tb[1]))

        if half == 0:
            cp2.wait()
            cp3 = pltpu.make_async_copy(x_hbm.at[row0 + 3], buf0, sem)
            cp3.start()

    pltpu.sync_copy(outbuf, out_hbm.at[pl.ds(row0, ROWS_PER_W)])


def kernel(x):
    mesh = plsc.VectorSubcoreMesh(core_axis_name="c", subcore_axis_name="s",
                                  num_cores=NC, num_subcores=NS)
    f = pl.kernel(
        _tec_body,
        out_type=jax.ShapeDtypeStruct((R, K), jnp.float32),
        mesh=mesh,
        compiler_params=pltpu.CompilerParams(needs_layout_passes=False),
        scratch_types=[
            pltpu.VMEM((N,), jnp.float32),
            pltpu.VMEM((N,), jnp.float32),
            pltpu.VMEM((N,), jnp.float32),
            pltpu.VMEM((ROWS_PER_W, K), jnp.float32),
            pltpu.VMEM((NGRP * L,), jnp.float32),
            pltpu.VMEM((NGRP * L,), jnp.float32),
            pltpu.SemaphoreType.DMA,
        ],
    )
    return f(x)
